# Initial kernel scaffold; baseline (speedup 1.0000x reference)
import jax
import jax.numpy as jnp
from jax.experimental import pallas as pl


def kernel(x, edge_index, W1, W2):
    raise NotImplementedError("write your pallas kernel here")


# speedup vs baseline: 6.8340x; 6.8340x over previous
import functools

import jax
import jax.numpy as jnp
from jax import lax
from jax.experimental import pallas as pl
from jax.experimental.pallas import tpu as pltpu
from jax.experimental.pallas import tpu_sc as plsc

NC = 2
NS = 16
NW = NC * NS
K = 128


def _mm_body(x_ref, w_ref, o_ref):
    o_ref[...] = jnp.dot(x_ref[...], w_ref[...],
                         preferred_element_type=jnp.float32)


def _matmul(x, w):
    n = x.shape[0]
    return pl.pallas_call(
        _mm_body,
        out_shape=jax.ShapeDtypeStruct((n, w.shape[1]), jnp.float32),
    )(x, w)


def _mid_body(acc_ref, degp_ref, w2_ref, y2_ref, deg_ref):
    deg = jnp.maximum(degp_ref[0] + degp_ref[1], 1.0)
    agg = (acc_ref[0] + acc_ref[1]) / deg[:, None]
    h = jnp.maximum(agg, 0.0)
    y2_ref[...] = jnp.dot(h, w2_ref[...], preferred_element_type=jnp.float32)
    deg_ref[...] = deg


def _mid(acc_part, deg_part, w2):
    n = acc_part.shape[1]
    c = w2.shape[1]
    return pl.pallas_call(
        _mid_body,
        out_shape=[
            jax.ShapeDtypeStruct((n, c), jnp.float32),
            jax.ShapeDtypeStruct((n,), jnp.float32),
        ],
    )(acc_part, deg_part, w2)


def _final_body(acc_ref, deg_ref, o_ref):
    o_ref[...] = (acc_ref[0] + acc_ref[1]) / deg_ref[...][:, None]


def _final(acc_part, deg):
    n, c = acc_part.shape[1], acc_part.shape[2]
    return pl.pallas_call(
        _final_body,
        out_shape=jax.ShapeDtypeStruct((n, c), jnp.float32),
    )(acc_part, deg)


@functools.lru_cache(maxsize=None)
def _make_sc_agg(n, e, d, with_deg):
    assert e % K == 0
    n_chunks = e // K
    row_stride, row_ch = 640, 80
    assert n <= NS * row_stride and n % row_ch == 0

    mesh = plsc.VectorSubcoreMesh(core_axis_name="c", subcore_axis_name="s",
                                  num_cores=NC, num_subcores=NS)

    out_type = [jax.ShapeDtypeStruct((NC, n, d), jnp.float32)]
    scratch = [
        pltpu.VMEM_SHARED((n, d), jnp.float32),
        pltpu.VMEM((K,), jnp.int32),
        pltpu.VMEM((K,), jnp.int32),
        pltpu.VMEM((K, d), jnp.float32),
        pltpu.VMEM((row_ch, d), jnp.float32),
        pltpu.SemaphoreType.DMA,
    ]
    if with_deg:
        out_type.append(jax.ShapeDtypeStruct((NC * n,), jnp.float32))
        scratch.append(pltpu.VMEM_SHARED((n,), jnp.float32))
        scratch.append(pltpu.VMEM((K,), jnp.float32))
        scratch.append(pltpu.VMEM((row_ch,), jnp.float32))

    @functools.partial(pl.kernel, out_type=out_type, mesh=mesh,
                       scratch_types=scratch,
                       compiler_params=pltpu.CompilerParams(
                           use_tc_tiling_on_sc=(d % 128 == 0)))
    def sc_agg(*refs):
        if with_deg:
            (y_hbm, src_hbm, dst_hbm,
             acc_out, deg_out,
             acc_sh, src_v, dst_v, rows_v, zblk_v, sem,
             deg_sh, ones_v, zrow_v) = refs
        else:
            (y_hbm, src_hbm, dst_hbm,
             acc_out,
             acc_sh, src_v, dst_v, rows_v, zblk_v, sem) = refs

        c = lax.axis_index("c")
        s = lax.axis_index("s")
        wid = c * NS + s

        zero16 = jnp.zeros((16,), jnp.float32)

        def fill_zblk(i, _):
            zblk_v[i // (d // 16), pl.ds((i % (d // 16)) * 16, 16)] = zero16
            return 0
        lax.fori_loop(0, row_ch * d // 16, fill_zblk, 0)

        rbase = s * row_stride
        n_row_ch = jnp.clip(n - rbase, 0, row_stride) // row_ch

        def zero_acc(i, _):
            off = rbase + i * row_ch
            pltpu.sync_copy(zblk_v, acc_sh.at[pl.ds(off, row_ch), :])
            return 0
        lax.fori_loop(0, n_row_ch, zero_acc, 0)
        if with_deg:
            for j in range(row_ch // 16):
                zrow_v[pl.ds(j * 16, 16)] = zero16
            for j in range(K // 16):
                ones_v[pl.ds(j * 16, 16)] = jnp.ones((16,), jnp.float32)

            def zero_deg(i, _):
                off = rbase + i * row_ch
                pltpu.sync_copy(zrow_v, deg_sh.at[pl.ds(off, row_ch)])
                return 0
            lax.fori_loop(0, n_row_ch, zero_deg, 0)

        plsc.subcore_barrier()

        base_cnt = n_chunks // NW
        extra = n_chunks % NW
        my_cnt = base_cnt + jnp.where(wid < extra, 1, 0)

        def body(j, _):
            cid = wid + j * NW
            ebase = cid * K
            pltpu.sync_copy(src_hbm.at[pl.ds(ebase, K)], src_v)
            pltpu.sync_copy(dst_hbm.at[pl.ds(ebase, K)], dst_v)
            pltpu.async_copy(y_hbm.at[src_v], rows_v, sem).wait()
            pltpu.sync_copy(rows_v, acc_sh.at[dst_v], add=True)
            if with_deg:
                pltpu.sync_copy(ones_v, deg_sh.at[dst_v], add=True)
            return 0
        lax.fori_loop(0, my_cnt, body, 0)

        plsc.subcore_barrier()

        def read_acc(i, _):
            off = rbase + i * row_ch
            pltpu.sync_copy(acc_sh.at[pl.ds(off, row_ch), :],
                            acc_out.at[c, pl.ds(off, row_ch), :])
            return 0
        lax.fori_loop(0, n_row_ch, read_acc, 0)
        if with_deg:
            def read_deg(i, _):
                off = rbase + i * row_ch
                pltpu.sync_copy(deg_sh.at[pl.ds(off, row_ch)], zrow_v)
                pltpu.sync_copy(zrow_v, deg_out.at[pl.ds(c * n + off, row_ch)])
                return 0
            lax.fori_loop(0, n_row_ch, read_deg, 0)

    return sc_agg


def kernel(x, edge_index, W1, W2):
    n, d = x.shape
    h = W1.shape[1]
    c = W2.shape[1]
    e = edge_index.shape[1]
    src = edge_index[0]
    dst = edge_index[1]

    y1 = _matmul(x, W1)
    agg1 = _make_sc_agg(n, e, h, True)
    acc1, degp = agg1(y1, src, dst)
    degp = degp.reshape(NC, n)

    y2, deg = _mid(acc1, degp, W2)

    agg2 = _make_sc_agg(n, e, c, False)
    (acc2,) = agg2(y2, src, dst)

    return _final(acc2, deg)


# speedup vs baseline: 13.1468x; 1.9237x over previous
import functools

import jax
import jax.numpy as jnp
from jax import lax
from jax.experimental import pallas as pl
from jax.experimental.pallas import tpu as pltpu
from jax.experimental.pallas import tpu_sc as plsc

NC = 2
NS = 16
NW = NC * NS


def _mm_body(x_ref, w_ref, o_ref):
    o_ref[...] = jnp.dot(x_ref[...], w_ref[...],
                         preferred_element_type=jnp.float32)


def _matmul(x, w):
    n = x.shape[0]
    return pl.pallas_call(
        _mm_body,
        out_shape=jax.ShapeDtypeStruct((n, w.shape[1]), jnp.float32),
    )(x, w)


def _mid_body(acc_ref, degp_ref, w2_ref, y2_ref, deg_ref):
    deg = jnp.maximum(degp_ref[0] + degp_ref[1], 1.0)
    agg = (acc_ref[0] + acc_ref[1]) / deg[:, None]
    h = jnp.maximum(agg, 0.0)
    y2_ref[...] = jnp.dot(h, w2_ref[...], preferred_element_type=jnp.float32)
    deg_ref[...] = deg


def _mid(acc_part, deg_part, w2):
    n = acc_part.shape[1]
    c = w2.shape[1]
    return pl.pallas_call(
        _mid_body,
        out_shape=[
            jax.ShapeDtypeStruct((n, c), jnp.float32),
            jax.ShapeDtypeStruct((n,), jnp.float32),
        ],
    )(acc_part, deg_part, w2)


def _final_body(acc_ref, deg_ref, o_ref):
    o_ref[...] = (acc_ref[0] + acc_ref[1]) / deg_ref[...][:, None]


def _final(acc_part, deg):
    n, c = acc_part.shape[1], acc_part.shape[2]
    return pl.pallas_call(
        _final_body,
        out_shape=jax.ShapeDtypeStruct((n, c), jnp.float32),
    )(acc_part, deg)


@functools.lru_cache(maxsize=None)
def _make_sc_agg(n, e, d, with_deg):
    ch = 128
    nbuf = 2 if d >= 128 else 6
    n_chunks = e // ch
    base_cnt = n_chunks // NW
    extra = n_chunks % NW
    assert n_chunks * ch == e and base_cnt % nbuf == 0 and extra <= NW
    row_stride, row_ch = 640, 80
    assert n <= NS * row_stride and n % row_ch == 0

    mesh = plsc.VectorSubcoreMesh(core_axis_name="c", subcore_axis_name="s",
                                  num_cores=NC, num_subcores=NS)

    out_type = [jax.ShapeDtypeStruct((NC, n, d), jnp.float32)]
    scratch = (
        [pltpu.VMEM_SHARED((n, d), jnp.float32)]
        + [pltpu.VMEM((ch, d), jnp.float32)] * nbuf
        + [pltpu.VMEM((ch,), jnp.int32)] * nbuf
        + [pltpu.VMEM((ch,), jnp.int32)] * nbuf
        + [pltpu.SemaphoreType.DMA] * (2 * nbuf)
        + [pltpu.VMEM((row_ch, d), jnp.float32)]
    )
    if with_deg:
        out_type.append(jax.ShapeDtypeStruct((NC * n,), jnp.float32))
        scratch.append(pltpu.VMEM_SHARED((n,), jnp.float32))
        scratch.append(pltpu.VMEM((ch,), jnp.float32))
        scratch.append(pltpu.VMEM((row_ch,), jnp.float32))

    @functools.partial(pl.kernel, out_type=out_type, mesh=mesh,
                       scratch_types=scratch,
                       compiler_params=pltpu.CompilerParams(
                           use_tc_tiling_on_sc=(d % 128 == 0)))
    def sc_agg(*refs):
        if with_deg:
            y_hbm, src_hbm, dst_hbm, acc_out, deg_out, acc_sh, *rest = refs
            deg_sh, ones_v, zrow_v = rest[5 * nbuf + 1:]
        else:
            y_hbm, src_hbm, dst_hbm, acc_out, acc_sh, *rest = refs
        rows_v = rest[:nbuf]
        src_v = rest[nbuf:2 * nbuf]
        dst_v = rest[2 * nbuf:3 * nbuf]
        sem_i = rest[3 * nbuf:4 * nbuf]
        sem_g = rest[4 * nbuf:5 * nbuf]
        zblk_v = rest[5 * nbuf]

        c = lax.axis_index("c")
        s = lax.axis_index("s")
        wid = c * NS + s

        zero16 = jnp.zeros((16,), jnp.float32)
        dl = d // 16

        def fill_zblk(i, _):
            zblk_v[i // dl, pl.ds((i % dl) * 16, 16)] = zero16
            return 0
        lax.fori_loop(0, row_ch * dl, fill_zblk, 0)

        rbase = s * row_stride
        n_row_ch = jnp.clip(n - rbase, 0, row_stride) // row_ch

        def zero_acc(i, _):
            off = rbase + i * row_ch
            pltpu.sync_copy(zblk_v, acc_sh.at[pl.ds(off, row_ch), :])
            return 0
        lax.fori_loop(0, n_row_ch, zero_acc, 0)
        if with_deg:
            for j in range(row_ch // 16):
                zrow_v[pl.ds(j * 16, 16)] = zero16
            for j in range(ch // 16):
                ones_v[pl.ds(j * 16, 16)] = jnp.ones((16,), jnp.float32)

            def zero_deg(i, _):
                off = rbase + i * row_ch
                pltpu.sync_copy(zrow_v, deg_sh.at[pl.ds(off, row_ch)])
                return 0
            lax.fori_loop(0, n_row_ch, zero_deg, 0)

        plsc.subcore_barrier()

        my_cnt = base_cnt + jnp.where(wid < extra, 1, 0)

        def idx_copies(j, b):
            ebase = (wid + j * NW) * ch
            return (
                pltpu.make_async_copy(src_hbm.at[pl.ds(ebase, ch)],
                                      src_v[b], sem_i[b]),
                pltpu.make_async_copy(dst_hbm.at[pl.ds(ebase, ch)],
                                      dst_v[b], sem_i[b]),
            )

        def gather(b):
            return pltpu.make_async_copy(y_hbm.at[src_v[b]], rows_v[b],
                                         sem_g[b])

        def scatter(b):
            pltpu.sync_copy(rows_v[b], acc_sh.at[dst_v[b]], add=True)
            if with_deg:
                pltpu.sync_copy(ones_v, deg_sh.at[dst_v[b]], add=True)

        for b in range(nbuf):
            for cp in idx_copies(b, b):
                cp.start()
        for cp in idx_copies(0, 0):
            cp.wait()
        gather(0).start()

        def outer(g, _):
            for b in range(nbuf):
                j = g * nbuf + b
                bn = (b + 1) % nbuf

                @pl.when(j + 1 < my_cnt)
                def _():
                    for cp in idx_copies(j + 1, bn):
                        cp.wait()
                    gather(bn).start()

                gather(b).wait()
                scatter(b)

                @pl.when(j + nbuf < my_cnt)
                def _():
                    for cp in idx_copies(j + nbuf, b):
                        cp.start()
            return 0
        lax.fori_loop(0, base_cnt // nbuf, outer, 0)

        bt = base_cnt % nbuf

        @pl.when(wid < extra)
        def _():
            gather(bt).wait()
            scatter(bt)

        plsc.subcore_barrier()

        def read_acc(i, _):
            off = rbase + i * row_ch
            pltpu.sync_copy(acc_sh.at[pl.ds(off, row_ch), :],
                            acc_out.at[c, pl.ds(off, row_ch), :])
            return 0
        lax.fori_loop(0, n_row_ch, read_acc, 0)
        if with_deg:
            def read_deg(i, _):
                off = rbase + i * row_ch
                pltpu.sync_copy(deg_sh.at[pl.ds(off, row_ch)], zrow_v)
                pltpu.sync_copy(zrow_v, deg_out.at[pl.ds(c * n + off, row_ch)])
                return 0
            lax.fori_loop(0, n_row_ch, read_deg, 0)

    return sc_agg


def kernel(x, edge_index, W1, W2):
    n, d = x.shape
    h = W1.shape[1]
    c = W2.shape[1]
    e = edge_index.shape[1]
    src = edge_index[0]
    dst = edge_index[1]

    y1 = _matmul(x, W1)
    agg1 = _make_sc_agg(n, e, h, True)
    acc1, degp = agg1(y1, src, dst)
    degp = degp.reshape(NC, n)

    y2, deg = _mid(acc1, degp, W2)

    agg2 = _make_sc_agg(n, e, c, False)
    (acc2,) = agg2(y2, src, dst)

    return _final(acc2, deg)


# speedup vs baseline: 13.3637x; 1.0165x over previous
import functools

import jax
import jax.numpy as jnp
from jax import lax
from jax.experimental import pallas as pl
from jax.experimental.pallas import tpu as pltpu
from jax.experimental.pallas import tpu_sc as plsc

NC = 2
NS = 16
NW = NC * NS


def _mm_body(x_ref, w_ref, o_ref):
    o_ref[...] = jnp.dot(x_ref[...], w_ref[...],
                         preferred_element_type=jnp.float32)


def _matmul(x, w):
    n = x.shape[0]
    return pl.pallas_call(
        _mm_body,
        out_shape=jax.ShapeDtypeStruct((n, w.shape[1]), jnp.float32),
    )(x, w)


def _mid_body(acc_ref, degp_ref, w2_ref, y2_ref, deg_ref):
    deg = jnp.maximum(degp_ref[0] + degp_ref[1], 1.0)
    agg = (acc_ref[0] + acc_ref[1]) / deg[:, None]
    h = jnp.maximum(agg, 0.0)
    y2_ref[...] = jnp.dot(h, w2_ref[...], preferred_element_type=jnp.float32)
    deg_ref[...] = deg


def _mid(acc_part, deg_part, w2):
    n = acc_part.shape[1]
    c = w2.shape[1]
    return pl.pallas_call(
        _mid_body,
        out_shape=[
            jax.ShapeDtypeStruct((n, c), jnp.float32),
            jax.ShapeDtypeStruct((n,), jnp.float32),
        ],
    )(acc_part, deg_part, w2)


def _final_body(acc_ref, deg_ref, o_ref):
    o_ref[...] = (acc_ref[0] + acc_ref[1]) / deg_ref[...][:, None]


def _final(acc_part, deg):
    n, c = acc_part.shape[1], acc_part.shape[2]
    return pl.pallas_call(
        _final_body,
        out_shape=jax.ShapeDtypeStruct((n, c), jnp.float32),
    )(acc_part, deg)


@functools.lru_cache(maxsize=None)
def _make_sc_agg(n, e, d, with_deg):
    ch = 128
    nbuf = 2 if d >= 128 else 6
    n_chunks = e // ch
    base_cnt = n_chunks // NW
    extra = n_chunks % NW
    assert n_chunks * ch == e and base_cnt % nbuf == 0 and extra <= NW
    row_stride, row_ch = 640, 80
    assert n <= NS * row_stride and n % row_ch == 0

    mesh = plsc.VectorSubcoreMesh(core_axis_name="c", subcore_axis_name="s",
                                  num_cores=NC, num_subcores=NS)

    out_type = [jax.ShapeDtypeStruct((NC, n, d), jnp.float32)]
    scratch = (
        [pltpu.VMEM_SHARED((n, d), jnp.float32)]
        + [pltpu.VMEM((ch, d), jnp.float32)] * nbuf
        + [pltpu.VMEM((ch,), jnp.int32)] * nbuf
        + [pltpu.VMEM((ch,), jnp.int32)] * nbuf
        + [pltpu.SemaphoreType.DMA] * (2 * nbuf)
        + [pltpu.VMEM((row_ch, d), jnp.float32)]
    )
    if with_deg:
        out_type.append(jax.ShapeDtypeStruct((NC * n,), jnp.float32))
        scratch.append(pltpu.VMEM_SHARED((n,), jnp.float32))
        scratch.append(pltpu.VMEM((ch,), jnp.float32))
        scratch.append(pltpu.VMEM((row_ch,), jnp.float32))
        scratch.append(pltpu.SemaphoreType.DMA)

    @functools.partial(pl.kernel, out_type=out_type, mesh=mesh,
                       scratch_types=scratch,
                       compiler_params=pltpu.CompilerParams(
                           use_tc_tiling_on_sc=(d % 128 == 0)))
    def sc_agg(*refs):
        if with_deg:
            y_hbm, src_hbm, dst_hbm, acc_out, deg_out, acc_sh, *rest = refs
            deg_sh, ones_v, zrow_v, sem_d = rest[5 * nbuf + 1:]
        else:
            y_hbm, src_hbm, dst_hbm, acc_out, acc_sh, *rest = refs
        rows_v = rest[:nbuf]
        src_v = rest[nbuf:2 * nbuf]
        dst_v = rest[2 * nbuf:3 * nbuf]
        sem_i = rest[3 * nbuf:4 * nbuf]
        sem_g = rest[4 * nbuf:5 * nbuf]
        zblk_v = rest[5 * nbuf]

        c = lax.axis_index("c")
        s = lax.axis_index("s")
        wid = c * NS + s

        zero16 = jnp.zeros((16,), jnp.float32)
        dl = d // 16

        def fill_zblk(i, _):
            zblk_v[i // dl, pl.ds((i % dl) * 16, 16)] = zero16
            return 0
        lax.fori_loop(0, row_ch * dl, fill_zblk, 0)

        rbase = s * row_stride
        n_row_ch = jnp.clip(n - rbase, 0, row_stride) // row_ch

        def acc_cp(i, out=False):
            off = rbase + i * row_ch
            dst = acc_sh.at[pl.ds(off, row_ch), :]
            if out:
                return pltpu.make_async_copy(
                    dst, acc_out.at[c, pl.ds(off, row_ch), :], sem_g[0])
            return pltpu.make_async_copy(zblk_v, dst, sem_g[0])

        def zero_acc(i, _):
            acc_cp(i).start()
            return 0
        lax.fori_loop(0, n_row_ch, zero_acc, 0)

        def zero_acc_w(i, _):
            acc_cp(0).wait()
            return 0
        lax.fori_loop(0, n_row_ch, zero_acc_w, 0)
        if with_deg:
            for j in range(row_ch // 16):
                zrow_v[pl.ds(j * 16, 16)] = zero16
            for j in range(ch // 16):
                ones_v[pl.ds(j * 16, 16)] = jnp.ones((16,), jnp.float32)

            def zero_deg(i, _):
                off = rbase + i * row_ch
                pltpu.sync_copy(zrow_v, deg_sh.at[pl.ds(off, row_ch)])
                return 0
            lax.fori_loop(0, n_row_ch, zero_deg, 0)

        plsc.subcore_barrier()

        my_cnt = base_cnt + jnp.where(wid < extra, 1, 0)

        def idx_copies(j, b):
            ebase = (wid + j * NW) * ch
            return (
                pltpu.make_async_copy(src_hbm.at[pl.ds(ebase, ch)],
                                      src_v[b], sem_i[b]),
                pltpu.make_async_copy(dst_hbm.at[pl.ds(ebase, ch)],
                                      dst_v[b], sem_i[b]),
            )

        def gather(b):
            return pltpu.make_async_copy(y_hbm.at[src_v[b]], rows_v[b],
                                         sem_g[b])

        def scatter(b):
            if with_deg:
                dsc = pltpu.async_copy(ones_v, deg_sh.at[dst_v[b]],
                                       sem_d, add=True)
                pltpu.sync_copy(rows_v[b], acc_sh.at[dst_v[b]], add=True)
                dsc.wait()
            else:
                pltpu.sync_copy(rows_v[b], acc_sh.at[dst_v[b]], add=True)

        for b in range(nbuf):
            for cp in idx_copies(b, b):
                cp.start()
        for cp in idx_copies(0, 0):
            cp.wait()
        gather(0).start()

        def outer(g, _):
            for b in range(nbuf):
                j = g * nbuf + b
                bn = (b + 1) % nbuf

                @pl.when(j + 1 < my_cnt)
                def _():
                    for cp in idx_copies(j + 1, bn):
                        cp.wait()
                    gather(bn).start()

                gather(b).wait()
                scatter(b)

                @pl.when(j + nbuf < my_cnt)
                def _():
                    for cp in idx_copies(j + nbuf, b):
                        cp.start()
            return 0
        lax.fori_loop(0, base_cnt // nbuf, outer, 0)

        bt = base_cnt % nbuf

        @pl.when(wid < extra)
        def _():
            gather(bt).wait()
            scatter(bt)

        plsc.subcore_barrier()

        def read_acc(i, _):
            acc_cp(i, out=True).start()
            return 0
        lax.fori_loop(0, n_row_ch, read_acc, 0)

        def read_acc_w(i, _):
            acc_cp(0, out=True).wait()
            return 0
        lax.fori_loop(0, n_row_ch, read_acc_w, 0)
        if with_deg:
            def read_deg(i, _):
                off = rbase + i * row_ch
                pltpu.sync_copy(deg_sh.at[pl.ds(off, row_ch)], zrow_v)
                pltpu.sync_copy(zrow_v, deg_out.at[pl.ds(c * n + off, row_ch)])
                return 0
            lax.fori_loop(0, n_row_ch, read_deg, 0)

    return sc_agg


def kernel(x, edge_index, W1, W2):
    n, d = x.shape
    h = W1.shape[1]
    c = W2.shape[1]
    e = edge_index.shape[1]
    src = edge_index[0]
    dst = edge_index[1]

    y1 = _matmul(x, W1)
    agg1 = _make_sc_agg(n, e, h, True)
    acc1, degp = agg1(y1, src, dst)
    degp = degp.reshape(NC, n)

    y2, deg = _mid(acc1, degp, W2)

    agg2 = _make_sc_agg(n, e, c, False)
    (acc2,) = agg2(y2, src, dst)

    return _final(acc2, deg)


# speedup vs baseline: 14.3644x; 1.0749x over previous
import functools

import jax
import jax.numpy as jnp
from jax import lax
from jax.experimental import pallas as pl
from jax.experimental.pallas import tpu as pltpu
from jax.experimental.pallas import tpu_sc as plsc

NC = 2
NS = 16
NW = NC * NS


def _mm_body(x_ref, w_ref, o_ref):
    o_ref[...] = jnp.dot(x_ref[...], w_ref[...],
                         preferred_element_type=jnp.float32)


def _matmul(x, w):
    n = x.shape[0]
    return pl.pallas_call(
        _mm_body,
        out_shape=jax.ShapeDtypeStruct((n, w.shape[1]), jnp.float32),
    )(x, w)


def _mid_body(acc_ref, degp_ref, w2_ref, y2_ref, deg_ref):
    deg = jnp.maximum(degp_ref[0] + degp_ref[1], 1.0)
    agg = (acc_ref[0] + acc_ref[1]) / deg[:, None]
    h = jnp.maximum(agg, 0.0)
    y2_ref[...] = jnp.dot(h, w2_ref[...], preferred_element_type=jnp.float32)
    deg_ref[...] = deg


def _mid(acc_part, deg_part, w2):
    n = acc_part.shape[1]
    c = w2.shape[1]
    return pl.pallas_call(
        _mid_body,
        out_shape=[
            jax.ShapeDtypeStruct((n, c), jnp.float32),
            jax.ShapeDtypeStruct((n,), jnp.float32),
        ],
    )(acc_part, deg_part, w2)


def _final_body(acc_ref, deg_ref, o_ref):
    o_ref[...] = (acc_ref[0] + acc_ref[1]) / deg_ref[...][:, None]


def _final(acc_part, deg):
    n, c = acc_part.shape[1], acc_part.shape[2]
    return pl.pallas_call(
        _final_body,
        out_shape=jax.ShapeDtypeStruct((n, c), jnp.float32),
    )(acc_part, deg)


@functools.lru_cache(maxsize=None)
def _make_sc_agg(n, e, d, with_deg):
    ch = 128
    nbuf = 2 if d >= 128 else 6
    n_chunks = e // ch
    base_cnt = n_chunks // NW
    extra = n_chunks % NW
    assert n_chunks * ch == e and base_cnt % nbuf == 0 and extra <= NW
    row_stride, row_ch = 640, 80
    assert n <= NS * row_stride and n % row_ch == 0

    mesh = plsc.VectorSubcoreMesh(core_axis_name="c", subcore_axis_name="s",
                                  num_cores=NC, num_subcores=NS)

    out_type = [jax.ShapeDtypeStruct((NC, n, d), jnp.float32)]
    scratch = (
        [pltpu.VMEM_SHARED((n, d), jnp.float32)]
        + [pltpu.VMEM((ch, d), jnp.float32)] * nbuf
        + [pltpu.VMEM((ch,), jnp.int32)] * nbuf
        + [pltpu.VMEM((ch,), jnp.int32)] * nbuf
        + [pltpu.VMEM((ch,), jnp.int32)] * nbuf
        + [pltpu.SemaphoreType.DMA] * (3 * nbuf)
        + [pltpu.VMEM((row_ch, d), jnp.float32)]
    )
    if with_deg:
        out_type.append(jax.ShapeDtypeStruct((NC * n,), jnp.float32))
        scratch.append(pltpu.VMEM_SHARED((n,), jnp.float32))
        scratch.append(pltpu.VMEM((ch,), jnp.float32))
        scratch.append(pltpu.VMEM((row_ch,), jnp.float32))
        scratch.extend([pltpu.SemaphoreType.DMA] * nbuf)

    @functools.partial(pl.kernel, out_type=out_type, mesh=mesh,
                       scratch_types=scratch,
                       compiler_params=pltpu.CompilerParams(
                           use_tc_tiling_on_sc=(d % 128 == 0)))
    def sc_agg(*refs):
        if with_deg:
            y_hbm, src_hbm, dst_hbm, acc_out, deg_out, acc_sh, *rest = refs
            tail_refs = rest[7 * nbuf + 1:]
            deg_sh, ones_v, zrow_v = tail_refs[:3]
            sem_d = tail_refs[3:]
        else:
            y_hbm, src_hbm, dst_hbm, acc_out, acc_sh, *rest = refs
        rows_v = rest[:nbuf]
        src_v = rest[nbuf:2 * nbuf]
        dst_v = rest[2 * nbuf:3 * nbuf]
        dst_s = rest[3 * nbuf:4 * nbuf]
        sem_i = rest[4 * nbuf:5 * nbuf]
        sem_g = rest[5 * nbuf:6 * nbuf]
        sem_s = rest[6 * nbuf:7 * nbuf]
        zblk_v = rest[7 * nbuf]

        c = lax.axis_index("c")
        s = lax.axis_index("s")
        wid = c * NS + s

        zero16 = jnp.zeros((16,), jnp.float32)
        dl = d // 16

        def fill_zblk(i, _):
            zblk_v[i // dl, pl.ds((i % dl) * 16, 16)] = zero16
            return 0
        lax.fori_loop(0, row_ch * dl, fill_zblk, 0)

        rbase = s * row_stride
        n_row_ch = jnp.clip(n - rbase, 0, row_stride) // row_ch

        def acc_cp(i, out=False):
            off = rbase + i * row_ch
            dst = acc_sh.at[pl.ds(off, row_ch), :]
            if out:
                return pltpu.make_async_copy(
                    dst, acc_out.at[c, pl.ds(off, row_ch), :], sem_g[0])
            return pltpu.make_async_copy(zblk_v, dst, sem_g[0])

        def zero_acc(i, _):
            acc_cp(i).start()
            return 0
        lax.fori_loop(0, n_row_ch, zero_acc, 0)

        def zero_acc_w(i, _):
            acc_cp(0).wait()
            return 0
        lax.fori_loop(0, n_row_ch, zero_acc_w, 0)
        if with_deg:
            for j in range(row_ch // 16):
                zrow_v[pl.ds(j * 16, 16)] = zero16
            for j in range(ch // 16):
                ones_v[pl.ds(j * 16, 16)] = jnp.ones((16,), jnp.float32)

            def zero_deg(i, _):
                off = rbase + i * row_ch
                pltpu.sync_copy(zrow_v, deg_sh.at[pl.ds(off, row_ch)])
                return 0
            lax.fori_loop(0, n_row_ch, zero_deg, 0)

        plsc.subcore_barrier()

        my_cnt = base_cnt + jnp.where(wid < extra, 1, 0)

        def idx_copies(j, b):
            ebase = (wid + j * NW) * ch
            return (
                pltpu.make_async_copy(src_hbm.at[pl.ds(ebase, ch)],
                                      src_v[b], sem_i[b]),
                pltpu.make_async_copy(dst_hbm.at[pl.ds(ebase, ch)],
                                      dst_v[b], sem_i[b]),
            )

        def gather(b):
            return pltpu.make_async_copy(y_hbm.at[src_v[b]], rows_v[b],
                                         sem_g[b])

        def scatter(b):
            for t in range(ch // 16):
                dst_s[b][pl.ds(t * 16, 16)] = dst_v[b][pl.ds(t * 16, 16)]
            pltpu.async_copy(rows_v[b], acc_sh.at[dst_s[b]], sem_s[b],
                             add=True)
            if with_deg:
                pltpu.async_copy(ones_v, deg_sh.at[dst_s[b]], sem_d[b],
                                 add=True)

        def scatter_wait(b):
            pltpu.make_async_copy(rows_v[b], acc_sh.at[dst_s[b]],
                                  sem_s[b]).wait()
            if with_deg:
                pltpu.make_async_copy(ones_v, deg_sh.at[dst_s[b]],
                                      sem_d[b]).wait()

        for b in range(nbuf):
            for cp in idx_copies(b, b):
                cp.start()
        for cp in idx_copies(0, 0):
            cp.wait()
        gather(0).start()

        def outer(g, _):
            for b in range(nbuf):
                j = g * nbuf + b
                bn = (b + 1) % nbuf

                @pl.when(j + 1 < my_cnt)
                def _():
                    for cp in idx_copies(j + 1, bn):
                        cp.wait()

                    @pl.when(j + 1 >= nbuf)
                    def _():
                        scatter_wait(bn)
                    gather(bn).start()

                gather(b).wait()
                scatter(b)

                @pl.when(j + nbuf < my_cnt)
                def _():
                    for cp in idx_copies(j + nbuf, b):
                        cp.start()
            return 0
        lax.fori_loop(0, base_cnt // nbuf, outer, 0)

        bt = base_cnt % nbuf

        @pl.when(wid < extra)
        def _():
            gather(bt).wait()
            scatter(bt)

        for b in range(nbuf):
            scatter_wait(b)

        plsc.subcore_barrier()

        def read_acc(i, _):
            acc_cp(i, out=True).start()
            return 0
        lax.fori_loop(0, n_row_ch, read_acc, 0)

        def read_acc_w(i, _):
            acc_cp(0, out=True).wait()
            return 0
        lax.fori_loop(0, n_row_ch, read_acc_w, 0)
        if with_deg:
            def read_deg(i, _):
                off = rbase + i * row_ch
                pltpu.sync_copy(deg_sh.at[pl.ds(off, row_ch)], zrow_v)
                pltpu.sync_copy(zrow_v, deg_out.at[pl.ds(c * n + off, row_ch)])
                return 0
            lax.fori_loop(0, n_row_ch, read_deg, 0)

    return sc_agg


def kernel(x, edge_index, W1, W2):
    n, d = x.shape
    h = W1.shape[1]
    c = W2.shape[1]
    e = edge_index.shape[1]
    src = edge_index[0]
    dst = edge_index[1]

    y1 = _matmul(x, W1)
    agg1 = _make_sc_agg(n, e, h, True)
    acc1, degp = agg1(y1, src, dst)
    degp = degp.reshape(NC, n)

    y2, deg = _mid(acc1, degp, W2)

    agg2 = _make_sc_agg(n, e, c, False)
    (acc2,) = agg2(y2, src, dst)

    return _final(acc2, deg)


# speedup vs baseline: 14.7189x; 1.0247x over previous
import functools

import jax
import jax.numpy as jnp
from jax import lax
from jax.experimental import pallas as pl
from jax.experimental.pallas import tpu as pltpu
from jax.experimental.pallas import tpu_sc as plsc

NC = 2
NS = 16
NW = NC * NS


def _mid_body(acc_ref, degp_ref, w1_ref, w2_ref, y2_ref, deg_ref):
    deg = jnp.maximum(degp_ref[0] + degp_ref[1], 1.0)
    agg = (acc_ref[0] + acc_ref[1]) / deg[:, None]
    h = jnp.maximum(
        jnp.dot(agg, w1_ref[...], preferred_element_type=jnp.float32), 0.0)
    y2_ref[...] = jnp.dot(h, w2_ref[...], preferred_element_type=jnp.float32)
    deg_ref[...] = deg


def _mid(acc_part, deg_part, w1, w2):
    n = acc_part.shape[1]
    c = w2.shape[1]
    return pl.pallas_call(
        _mid_body,
        out_shape=[
            jax.ShapeDtypeStruct((n, c), jnp.float32),
            jax.ShapeDtypeStruct((n,), jnp.float32),
        ],
    )(acc_part, deg_part, w1, w2)


def _final_body(acc_ref, deg_ref, o_ref):
    o_ref[...] = (acc_ref[0] + acc_ref[1]) / deg_ref[...][:, None]


def _final(acc_part, deg):
    n, c = acc_part.shape[1], acc_part.shape[2]
    return pl.pallas_call(
        _final_body,
        out_shape=jax.ShapeDtypeStruct((n, c), jnp.float32),
    )(acc_part, deg)


@functools.lru_cache(maxsize=None)
def _make_sc_agg(n, e, d, with_deg):
    ch = 128
    nbuf = 2 if d >= 128 else 6
    n_chunks = e // ch
    base_cnt = n_chunks // NW
    extra = n_chunks % NW
    assert n_chunks * ch == e and base_cnt % nbuf == 0 and extra <= NW
    row_stride, row_ch = 640, 80
    assert n <= NS * row_stride and n % row_ch == 0

    mesh = plsc.VectorSubcoreMesh(core_axis_name="c", subcore_axis_name="s",
                                  num_cores=NC, num_subcores=NS)

    out_type = [jax.ShapeDtypeStruct((NC, n, d), jnp.float32)]
    scratch = (
        [pltpu.VMEM_SHARED((n, d), jnp.float32)]
        + [pltpu.VMEM((ch, d), jnp.float32)] * nbuf
        + [pltpu.VMEM((ch,), jnp.int32)] * nbuf
        + [pltpu.VMEM((ch,), jnp.int32)] * nbuf
        + [pltpu.VMEM((ch,), jnp.int32)] * nbuf
        + [pltpu.SemaphoreType.DMA] * (3 * nbuf)
        + [pltpu.VMEM((row_ch, d), jnp.float32)]
    )
    if with_deg:
        out_type.append(jax.ShapeDtypeStruct((NC * n,), jnp.float32))
        scratch.append(pltpu.VMEM_SHARED((n,), jnp.float32))
        scratch.append(pltpu.VMEM((ch,), jnp.float32))
        scratch.append(pltpu.VMEM((row_ch,), jnp.float32))
        scratch.extend([pltpu.SemaphoreType.DMA] * nbuf)

    @functools.partial(pl.kernel, out_type=out_type, mesh=mesh,
                       scratch_types=scratch,
                       compiler_params=pltpu.CompilerParams(
                           use_tc_tiling_on_sc=(d % 128 == 0)))
    def sc_agg(*refs):
        if with_deg:
            y_hbm, src_hbm, dst_hbm, acc_out, deg_out, acc_sh, *rest = refs
            tail_refs = rest[7 * nbuf + 1:]
            deg_sh, ones_v, zrow_v = tail_refs[:3]
            sem_d = tail_refs[3:]
        else:
            y_hbm, src_hbm, dst_hbm, acc_out, acc_sh, *rest = refs
        rows_v = rest[:nbuf]
        src_v = rest[nbuf:2 * nbuf]
        dst_v = rest[2 * nbuf:3 * nbuf]
        dst_s = rest[3 * nbuf:4 * nbuf]
        sem_i = rest[4 * nbuf:5 * nbuf]
        sem_g = rest[5 * nbuf:6 * nbuf]
        sem_s = rest[6 * nbuf:7 * nbuf]
        zblk_v = rest[7 * nbuf]

        c = lax.axis_index("c")
        s = lax.axis_index("s")
        wid = c * NS + s

        zero16 = jnp.zeros((16,), jnp.float32)
        dl = d // 16

        def fill_zblk(i, _):
            zblk_v[i // dl, pl.ds((i % dl) * 16, 16)] = zero16
            return 0
        lax.fori_loop(0, row_ch * dl, fill_zblk, 0)

        rbase = s * row_stride
        n_row_ch = jnp.clip(n - rbase, 0, row_stride) // row_ch

        def acc_cp(i, out=False):
            off = rbase + i * row_ch
            dst = acc_sh.at[pl.ds(off, row_ch), :]
            if out:
                return pltpu.make_async_copy(
                    dst, acc_out.at[c, pl.ds(off, row_ch), :], sem_g[0])
            return pltpu.make_async_copy(zblk_v, dst, sem_g[0])

        def zero_acc(i, _):
            acc_cp(i).start()
            return 0
        lax.fori_loop(0, n_row_ch, zero_acc, 0)

        def zero_acc_w(i, _):
            acc_cp(0).wait()
            return 0
        lax.fori_loop(0, n_row_ch, zero_acc_w, 0)
        if with_deg:
            for j in range(row_ch // 16):
                zrow_v[pl.ds(j * 16, 16)] = zero16
            for j in range(ch // 16):
                ones_v[pl.ds(j * 16, 16)] = jnp.ones((16,), jnp.float32)

            def zero_deg(i, _):
                off = rbase + i * row_ch
                pltpu.sync_copy(zrow_v, deg_sh.at[pl.ds(off, row_ch)])
                return 0
            lax.fori_loop(0, n_row_ch, zero_deg, 0)

        plsc.subcore_barrier()

        my_cnt = base_cnt + jnp.where(wid < extra, 1, 0)

        def idx_copies(j, b):
            ebase = (wid + j * NW) * ch
            return (
                pltpu.make_async_copy(src_hbm.at[pl.ds(ebase, ch)],
                                      src_v[b], sem_i[b]),
                pltpu.make_async_copy(dst_hbm.at[pl.ds(ebase, ch)],
                                      dst_v[b], sem_i[b]),
            )

        def gather(b):
            return pltpu.make_async_copy(y_hbm.at[src_v[b]], rows_v[b],
                                         sem_g[b])

        def scatter(b):
            for t in range(ch // 16):
                dst_s[b][pl.ds(t * 16, 16)] = dst_v[b][pl.ds(t * 16, 16)]
            pltpu.async_copy(rows_v[b], acc_sh.at[dst_s[b]], sem_s[b],
                             add=True)
            if with_deg:
                pltpu.async_copy(ones_v, deg_sh.at[dst_s[b]], sem_d[b],
                                 add=True)

        def scatter_wait(b):
            pltpu.make_async_copy(rows_v[b], acc_sh.at[dst_s[b]],
                                  sem_s[b]).wait()
            if with_deg:
                pltpu.make_async_copy(ones_v, deg_sh.at[dst_s[b]],
                                      sem_d[b]).wait()

        for b in range(nbuf):
            for cp in idx_copies(b, b):
                cp.start()
        for cp in idx_copies(0, 0):
            cp.wait()
        gather(0).start()

        def outer(g, _):
            for b in range(nbuf):
                j = g * nbuf + b
                bn = (b + 1) % nbuf

                @pl.when(j + 1 < my_cnt)
                def _():
                    for cp in idx_copies(j + 1, bn):
                        cp.wait()

                    @pl.when(j + 1 >= nbuf)
                    def _():
                        scatter_wait(bn)
                    gather(bn).start()

                gather(b).wait()
                scatter(b)

                @pl.when(j + nbuf < my_cnt)
                def _():
                    for cp in idx_copies(j + nbuf, b):
                        cp.start()
            return 0
        lax.fori_loop(0, base_cnt // nbuf, outer, 0)

        bt = base_cnt % nbuf

        @pl.when(wid < extra)
        def _():
            gather(bt).wait()
            scatter(bt)

        for b in range(nbuf):
            scatter_wait(b)

        plsc.subcore_barrier()

        def read_acc(i, _):
            acc_cp(i, out=True).start()
            return 0
        lax.fori_loop(0, n_row_ch, read_acc, 0)

        def read_acc_w(i, _):
            acc_cp(0, out=True).wait()
            return 0
        lax.fori_loop(0, n_row_ch, read_acc_w, 0)
        if with_deg:
            def read_deg(i, _):
                off = rbase + i * row_ch
                pltpu.sync_copy(deg_sh.at[pl.ds(off, row_ch)], zrow_v)
                pltpu.sync_copy(zrow_v, deg_out.at[pl.ds(c * n + off, row_ch)])
                return 0
            lax.fori_loop(0, n_row_ch, read_deg, 0)

    return sc_agg


def kernel(x, edge_index, W1, W2):
    n, d = x.shape
    h = W1.shape[1]
    c = W2.shape[1]
    e = edge_index.shape[1]
    src = edge_index[0]
    dst = edge_index[1]

    agg1 = _make_sc_agg(n, e, d, True)
    acc1, degp = agg1(x, src, dst)
    degp = degp.reshape(NC, n)

    y2, deg = _mid(acc1, degp, W1, W2)

    agg2 = _make_sc_agg(n, e, c, False)
    (acc2,) = agg2(y2, src, dst)

    return _final(acc2, deg)


# speedup vs baseline: 14.8875x; 1.0115x over previous
import functools

import jax
import jax.numpy as jnp
from jax import lax
from jax.experimental import pallas as pl
from jax.experimental.pallas import tpu as pltpu
from jax.experimental.pallas import tpu_sc as plsc

NC = 2
NS = 16
NW = NC * NS


def _mid_body(acc_ref, degp_ref, w1_ref, w2_ref, y2_ref, deg_ref):
    deg = jnp.maximum(degp_ref[0] + degp_ref[1], 1.0)
    agg = (acc_ref[0] + acc_ref[1]) / deg[:, None]
    h = jnp.maximum(
        jnp.dot(agg, w1_ref[...], preferred_element_type=jnp.float32), 0.0)
    y2_ref[...] = jnp.dot(h, w2_ref[...], preferred_element_type=jnp.float32)
    deg_ref[...] = deg


def _mid(acc_part, deg_part, w1, w2):
    n = acc_part.shape[1]
    c = w2.shape[1]
    return pl.pallas_call(
        _mid_body,
        out_shape=[
            jax.ShapeDtypeStruct((n, c), jnp.float32),
            jax.ShapeDtypeStruct((n,), jnp.float32),
        ],
    )(acc_part, deg_part, w1, w2)


def _final_body(acc_ref, deg_ref, o_ref):
    o_ref[...] = (acc_ref[0] + acc_ref[1]) / deg_ref[...][:, None]


def _final(acc_part, deg):
    n, c = acc_part.shape[1], acc_part.shape[2]
    return pl.pallas_call(
        _final_body,
        out_shape=jax.ShapeDtypeStruct((n, c), jnp.float32),
    )(acc_part, deg)


@functools.lru_cache(maxsize=None)
def _make_sc_agg(n, e, d, with_deg):
    ch = 128
    nbuf = 2 if d >= 128 else 6
    n_chunks = e // ch
    base_cnt = n_chunks // NW
    extra = n_chunks % NW
    assert n_chunks * ch == e and base_cnt % nbuf == 0 and extra <= NW
    row_stride, row_ch = 640, 80
    assert n <= NS * row_stride and n % row_ch == 0

    mesh = plsc.VectorSubcoreMesh(core_axis_name="c", subcore_axis_name="s",
                                  num_cores=NC, num_subcores=NS)

    out_type = [jax.ShapeDtypeStruct((NC, n, d), jnp.float32)]
    scratch = (
        [pltpu.VMEM_SHARED((n, d), jnp.float32)]
        + [pltpu.VMEM((ch, d), jnp.float32)] * nbuf
        + [pltpu.VMEM((ch,), jnp.int32)] * nbuf
        + [pltpu.VMEM((ch,), jnp.int32)] * nbuf
        + [pltpu.VMEM((ch,), jnp.int32)] * nbuf
        + [pltpu.SemaphoreType.DMA] * (3 * nbuf)
        + [pltpu.VMEM((row_ch, d), jnp.float32)]
    )
    if with_deg:
        out_type.append(jax.ShapeDtypeStruct((NC * n,), jnp.float32))
        scratch.append(pltpu.VMEM_SHARED((n,), jnp.float32))
        scratch.append(pltpu.VMEM((ch,), jnp.float32))
        scratch.append(pltpu.VMEM((row_ch,), jnp.float32))
        scratch.extend([pltpu.SemaphoreType.DMA] * nbuf)

    @functools.partial(pl.kernel, out_type=out_type, mesh=mesh,
                       scratch_types=scratch,
                       compiler_params=pltpu.CompilerParams(
                           use_tc_tiling_on_sc=(d % 128 == 0)))
    def sc_agg(*refs):
        if with_deg:
            y_hbm, src_hbm, dst_hbm, acc_out, deg_out, acc_sh, *rest = refs
            tail_refs = rest[7 * nbuf + 1:]
            deg_sh, ones_v, zrow_v = tail_refs[:3]
            sem_d = tail_refs[3:]
        else:
            y_hbm, src_hbm, dst_hbm, acc_out, acc_sh, *rest = refs
        rows_v = rest[:nbuf]
        src_v = rest[nbuf:2 * nbuf]
        dst_v = rest[2 * nbuf:3 * nbuf]
        dst_s = rest[3 * nbuf:4 * nbuf]
        sem_i = rest[4 * nbuf:5 * nbuf]
        sem_g = rest[5 * nbuf:6 * nbuf]
        sem_s = rest[6 * nbuf:7 * nbuf]
        zblk_v = rest[7 * nbuf]

        c = lax.axis_index("c")
        s = lax.axis_index("s")
        wid = c * NS + s

        my_cnt = base_cnt + jnp.where(wid < extra, 1, 0)

        def idx_copies(j, b):
            ebase = (wid + j * NW) * ch
            return (
                pltpu.make_async_copy(src_hbm.at[pl.ds(ebase, ch)],
                                      src_v[b], sem_i[b]),
                pltpu.make_async_copy(dst_hbm.at[pl.ds(ebase, ch)],
                                      dst_v[b], sem_i[b]),
            )

        def gather(b):
            return pltpu.make_async_copy(y_hbm.at[src_v[b]], rows_v[b],
                                         sem_g[b])

        for b in range(nbuf):
            for cp in idx_copies(b, b):
                cp.start()

        zero16 = jnp.zeros((16,), jnp.float32)
        dl = d // 16

        def fill_zblk(i, _):
            zblk_v[i // dl, pl.ds((i % dl) * 16, 16)] = zero16
            return 0
        lax.fori_loop(0, row_ch * dl, fill_zblk, 0)

        rbase = s * row_stride
        n_row_ch = jnp.clip(n - rbase, 0, row_stride) // row_ch

        def acc_cp(i, out=False):
            off = rbase + i * row_ch
            dst = acc_sh.at[pl.ds(off, row_ch), :]
            if out:
                return pltpu.make_async_copy(
                    dst, acc_out.at[c, pl.ds(off, row_ch), :], sem_s[0])
            return pltpu.make_async_copy(zblk_v, dst, sem_s[0])

        def zero_acc(i, _):
            acc_cp(i).start()
            return 0
        lax.fori_loop(0, n_row_ch, zero_acc, 0)

        for cp in idx_copies(0, 0):
            cp.wait()
        gather(0).start()

        def zero_acc_w(i, _):
            acc_cp(0).wait()
            return 0
        lax.fori_loop(0, n_row_ch, zero_acc_w, 0)
        if with_deg:
            for j in range(row_ch // 16):
                zrow_v[pl.ds(j * 16, 16)] = zero16
            for j in range(ch // 16):
                ones_v[pl.ds(j * 16, 16)] = jnp.ones((16,), jnp.float32)

            def zero_deg(i, _):
                off = rbase + i * row_ch
                pltpu.sync_copy(zrow_v, deg_sh.at[pl.ds(off, row_ch)])
                return 0
            lax.fori_loop(0, n_row_ch, zero_deg, 0)

        plsc.subcore_barrier()

        def scatter(b):
            for t in range(ch // 16):
                dst_s[b][pl.ds(t * 16, 16)] = dst_v[b][pl.ds(t * 16, 16)]
            pltpu.async_copy(rows_v[b], acc_sh.at[dst_s[b]], sem_s[b],
                             add=True)
            if with_deg:
                pltpu.async_copy(ones_v, deg_sh.at[dst_s[b]], sem_d[b],
                                 add=True)

        def scatter_wait(b):
            pltpu.make_async_copy(rows_v[b], acc_sh.at[dst_s[b]],
                                  sem_s[b]).wait()
            if with_deg:
                pltpu.make_async_copy(ones_v, deg_sh.at[dst_s[b]],
                                      sem_d[b]).wait()

        def outer(g, _):
            for b in range(nbuf):
                j = g * nbuf + b
                bn = (b + 1) % nbuf

                @pl.when(j + 1 < my_cnt)
                def _():
                    for cp in idx_copies(j + 1, bn):
                        cp.wait()

                    @pl.when(j + 1 >= nbuf)
                    def _():
                        scatter_wait(bn)
                    gather(bn).start()

                gather(b).wait()
                scatter(b)

                @pl.when(j + nbuf < my_cnt)
                def _():
                    for cp in idx_copies(j + nbuf, b):
                        cp.start()
            return 0
        lax.fori_loop(0, base_cnt // nbuf, outer, 0)

        bt = base_cnt % nbuf

        @pl.when(wid < extra)
        def _():
            gather(bt).wait()
            scatter(bt)

        for b in range(nbuf):
            scatter_wait(b)

        plsc.subcore_barrier()

        def read_acc(i, _):
            acc_cp(i, out=True).start()
            return 0
        lax.fori_loop(0, n_row_ch, read_acc, 0)

        def read_acc_w(i, _):
            acc_cp(0, out=True).wait()
            return 0
        lax.fori_loop(0, n_row_ch, read_acc_w, 0)
        if with_deg:
            def read_deg(i, _):
                off = rbase + i * row_ch
                pltpu.sync_copy(deg_sh.at[pl.ds(off, row_ch)], zrow_v)
                pltpu.sync_copy(zrow_v, deg_out.at[pl.ds(c * n + off, row_ch)])
                return 0
            lax.fori_loop(0, n_row_ch, read_deg, 0)

    return sc_agg


def kernel(x, edge_index, W1, W2):
    n, d = x.shape
    h = W1.shape[1]
    c = W2.shape[1]
    e = edge_index.shape[1]
    src = edge_index[0]
    dst = edge_index[1]

    agg1 = _make_sc_agg(n, e, d, True)
    acc1, degp = agg1(x, src, dst)
    degp = degp.reshape(NC, n)

    y2, deg = _mid(acc1, degp, W1, W2)

    agg2 = _make_sc_agg(n, e, c, False)
    (acc2,) = agg2(y2, src, dst)

    return _final(acc2, deg)
